# baseline (speedup 1.0000x reference)
import functools

import jax
import jax.numpy as jnp
from jax import lax
from jax.experimental import pallas as pl
from jax.experimental.pallas import tpu as pltpu
from jax.experimental.pallas import tpu_sc as plsc

_HIDDEN = 128
_N_ROWS = 4096 * 50

_INFO = plsc.get_sparse_core_info()
_NC = _INFO.num_cores
_NS = _INFO.num_subcores
_NW = _NC * _NS
_PER_W = _N_ROWS // _NW
_CHUNK = 400
_N_CHUNKS = _PER_W // _CHUNK


def _sc_gather_body(idx_hbm, table_hbm, out_hbm, idx_v, rows_v, sem):
    wid = lax.axis_index("s") * _NC + lax.axis_index("c")
    base = wid * _PER_W

    def chunk(c, carry):
        off = base + c * _CHUNK
        pltpu.sync_copy(idx_hbm.at[pl.ds(off, _CHUNK)], idx_v)
        pltpu.async_copy(table_hbm.at[idx_v], rows_v, sem).wait()
        pltpu.sync_copy(rows_v, out_hbm.at[pl.ds(off, _CHUNK)])
        return carry

    lax.fori_loop(0, _N_CHUNKS, chunk, 0)


_sc_gather = functools.partial(
    pl.kernel,
    mesh=plsc.VectorSubcoreMesh(core_axis_name="c", subcore_axis_name="s"),
    out_type=jax.ShapeDtypeStruct((_N_ROWS, _HIDDEN), jnp.float32),
    scratch_types=[
        pltpu.VMEM((_CHUNK,), jnp.int32),
        pltpu.VMEM((_CHUNK, _HIDDEN), jnp.float32),
        pltpu.SemaphoreType.DMA,
    ],
    compiler_params=pltpu.CompilerParams(use_tc_tiling_on_sc=True),
)(_sc_gather_body)


_BLK = 2048


def _mlp_body(x_ref, w1_ref, b1_ref, w2_ref, b2_ref, o_ref):
    h = jnp.dot(x_ref[...], w1_ref[...], preferred_element_type=jnp.float32)
    h = jnp.maximum(h + b1_ref[...], 0.0)
    o = jnp.dot(h, w2_ref[...], preferred_element_type=jnp.float32)
    o_ref[...] = jnp.maximum(o + b2_ref[...], 0.0)


def _mlp(x, W1, b1, W2, b2):
    code = W2.shape[1]
    return pl.pallas_call(
        _mlp_body,
        grid=(_N_ROWS // _BLK,),
        in_specs=[
            pl.BlockSpec((_BLK, _HIDDEN), lambda i: (i, 0)),
            pl.BlockSpec((_HIDDEN, _HIDDEN), lambda i: (0, 0)),
            pl.BlockSpec((1, _HIDDEN), lambda i: (0, 0)),
            pl.BlockSpec((_HIDDEN, code), lambda i: (0, 0)),
            pl.BlockSpec((1, code), lambda i: (0, 0)),
        ],
        out_specs=pl.BlockSpec((_BLK, code), lambda i: (i, 0)),
        out_shape=jax.ShapeDtypeStruct((_N_ROWS, code), jnp.float32),
    )(x, W1, b1[None, :], W2, b2[None, :])


def kernel(src_seq, emb_table, W1, b1, W2, b2):
    B, L = src_seq.shape
    idx = src_seq.reshape(-1).astype(jnp.int32)
    gathered = _sc_gather(idx, emb_table)
    out = _mlp(gathered, W1, b1, W2, b2)
    return out.reshape(B, L, W2.shape[1])


# speedup vs baseline: 1.5618x; 1.5618x over previous
import functools

import jax
import jax.numpy as jnp
from jax import lax
from jax.experimental import pallas as pl
from jax.experimental.pallas import tpu as pltpu
from jax.experimental.pallas import tpu_sc as plsc

_HIDDEN = 128
_B = 4096
_L = 50
_N_ROWS = _B * _L

_INFO = plsc.get_sparse_core_info()
_NC = _INFO.num_cores
_NS = _INFO.num_subcores
_NW = _NC * _NS
_PER_W = _N_ROWS // _NW
_SEQ_PER_W = _PER_W // _L
_CHUNK = 640
_N_CHUNKS = _PER_W // _CHUNK
_SLABS = _CHUNK // _HIDDEN


def _sc_gather_body(idx_hbm, table_hbm, out_hbm, idx2_v, idx_v, rows_v, sem):
    wid = lax.axis_index("s") * _NC + lax.axis_index("c")

    pltpu.sync_copy(idx_hbm.at[pl.ds(wid * _SEQ_PER_W, _SEQ_PER_W), :], idx2_v)

    def flatten(j, carry):
        k = j * 16 + lax.iota(jnp.int32, 16)
        r = lax.shift_right_logical(k * 41944, 21)
        col = k - r * _L
        idx_v[pl.ds(j * 16, 16)] = plsc.load_gather(idx2_v, [r, col])
        return carry

    lax.fori_loop(0, _PER_W // 16, flatten, 0, unroll=8)

    def chunk(c, carry):
        off = c * _CHUNK
        pltpu.async_copy(
            table_hbm.at[idx_v.at[pl.ds(off, _CHUNK)]], rows_v, sem
        ).wait()
        pltpu.sync_copy(
            rows_v.reshape(_SLABS, _HIDDEN, _HIDDEN),
            out_hbm.at[pl.ds(wid * (_PER_W // _HIDDEN) + c * _SLABS, _SLABS)],
        )
        return carry

    lax.fori_loop(0, _N_CHUNKS, chunk, 0)


_sc_gather = functools.partial(
    pl.kernel,
    mesh=plsc.VectorSubcoreMesh(core_axis_name="c", subcore_axis_name="s"),
    out_type=jax.ShapeDtypeStruct((_N_ROWS // _HIDDEN, _HIDDEN, _HIDDEN),
                                  jnp.float32),
    scratch_types=[
        pltpu.VMEM((_SEQ_PER_W, _L), jnp.int32),
        pltpu.VMEM((_PER_W,), jnp.int32),
        pltpu.VMEM((_CHUNK, _HIDDEN), jnp.float32),
        pltpu.SemaphoreType.DMA,
    ],
    compiler_params=pltpu.CompilerParams(
        needs_layout_passes=False, use_tc_tiling_on_sc=True
    ),
)(_sc_gather_body)


_BB = 64
_BLK = _BB * _L


def _mlp_body(x_ref, w1_ref, b1_ref, w2_ref, b2_ref, o_ref):
    h = jnp.dot(x_ref[...], w1_ref[...], preferred_element_type=jnp.float32)
    h = jnp.maximum(h + b1_ref[...], 0.0)
    o = jnp.dot(h, w2_ref[...], preferred_element_type=jnp.float32)
    o_ref[...] = jnp.maximum(o + b2_ref[...], 0.0).reshape(o_ref.shape)


def _mlp(x, W1, b1, W2, b2):
    code = W2.shape[1]
    return pl.pallas_call(
        _mlp_body,
        grid=(_B // _BB,),
        in_specs=[
            pl.BlockSpec((_BLK, _HIDDEN), lambda i: (i, 0)),
            pl.BlockSpec((_HIDDEN, _HIDDEN), lambda i: (0, 0)),
            pl.BlockSpec((1, _HIDDEN), lambda i: (0, 0)),
            pl.BlockSpec((_HIDDEN, code), lambda i: (0, 0)),
            pl.BlockSpec((1, code), lambda i: (0, 0)),
        ],
        out_specs=pl.BlockSpec((_BB, _L, code), lambda i: (i, 0, 0)),
        out_shape=jax.ShapeDtypeStruct((_B, _L, code), jnp.float32),
    )(x, W1, b1[None, :], W2, b2[None, :])


def kernel(src_seq, emb_table, W1, b1, W2, b2):
    B, L = src_seq.shape
    gathered = _sc_gather(src_seq, emb_table).reshape(_N_ROWS, _HIDDEN)
    return _mlp(gathered, W1, b1, W2, b2)
